# Initial kernel scaffold; baseline (speedup 1.0000x reference)
import jax
import jax.numpy as jnp
from jax.experimental import pallas as pl


def kernel(hidden_states, true_indices):
    raise NotImplementedError("write your pallas kernel here")


# speedup vs baseline: 8.7691x; 8.7691x over previous
import functools

import jax
import jax.numpy as jnp
from jax.experimental import pallas as pl


def _masked_gelu_kernel(idx_ref, x_ref, o_ref, *, block_tokens: int):
    t = pl.program_id(0)
    base = t * block_tokens
    ids = base + jax.lax.broadcasted_iota(jnp.int32, (block_tokens, 1), 0)
    idx = idx_ref[:]
    hit = ids == idx[None, :]
    mask = jnp.any(hit, axis=1)
    x = x_ref[:]
    y = jax.nn.gelu(x, approximate=True)
    o_ref[:] = jnp.where(mask[None, :, None], y, x)


def kernel(hidden_states, true_indices):
    B, T, F = hidden_states.shape
    block_tokens = 128
    grid = (T // block_tokens,)
    fn = functools.partial(_masked_gelu_kernel, block_tokens=block_tokens)
    return pl.pallas_call(
        fn,
        grid=grid,
        in_specs=[
            pl.BlockSpec((true_indices.shape[0],), lambda t: (0,)),
            pl.BlockSpec((B, block_tokens, F), lambda t: (0, t, 0)),
        ],
        out_specs=pl.BlockSpec((B, block_tokens, F), lambda t: (0, t, 0)),
        out_shape=jax.ShapeDtypeStruct((B, T, F), hidden_states.dtype),
    )(true_indices, hidden_states)
